# Initial kernel scaffold; baseline (speedup 1.0000x reference)
import jax
import jax.numpy as jnp
from jax.experimental import pallas as pl


def kernel(x, edge_index, W, b):
    raise NotImplementedError("write your pallas kernel here")


# speedup vs baseline: 19.8596x; 19.8596x over previous
import functools

import jax
import jax.numpy as jnp
from jax import lax
from jax.experimental import pallas as pl
from jax.experimental.pallas import tpu as pltpu
from jax.experimental.pallas import tpu_sc as plsc

NC = 2
NS = 16
NW = NC * NS
LANES = 16
CHUNK = 128


def _make_deg_kernel(npad, e_t):
    mesh = plsc.VectorSubcoreMesh(core_axis_name="c", subcore_axis_name="s")

    @functools.partial(
        pl.kernel,
        out_type=jax.ShapeDtypeStruct((NW, npad), jnp.float32),
        mesh=mesh,
        scratch_types=[
            pltpu.VMEM((e_t,), jnp.int32),
            pltpu.VMEM((npad,), jnp.float32),
        ],
        compiler_params=pltpu.CompilerParams(needs_layout_passes=False),
    )
    def deg_kernel(dst_hbm, out_hbm, dst_v, deg_v):
        c = lax.axis_index("c")
        s = lax.axis_index("s")
        wid = s * NC + c
        zeros16 = jnp.zeros((LANES,), jnp.float32)

        def zero_body(i, _):
            deg_v[pl.ds(i * LANES, LANES)] = zeros16
            return 0

        lax.fori_loop(0, npad // LANES, zero_body, 0)
        pltpu.sync_copy(dst_hbm.at[pl.ds(wid * e_t, e_t)], dst_v)
        ones16 = jnp.ones((LANES,), jnp.float32)

        def body(i, _):
            idx = dst_v[pl.ds(i * LANES, LANES)]
            plsc.addupdate_scatter(deg_v, [idx], ones16)
            return 0

        lax.fori_loop(0, e_t // LANES, body, 0)
        pltpu.sync_copy(deg_v, out_hbm.at[wid])

    return deg_kernel


def _make_agg_kernel(nacc, d, n_chunks):
    mesh = plsc.VectorSubcoreMesh(core_axis_name="c", subcore_axis_name="s")
    rows_per_tile = nacc // NS

    @functools.partial(
        pl.kernel,
        out_type=jax.ShapeDtypeStruct((NC, nacc, d), jnp.float32),
        mesh=mesh,
        scratch_types=[
            pltpu.VMEM((n_chunks, CHUNK), jnp.int32),
            pltpu.VMEM((n_chunks, CHUNK), jnp.int32),
            pltpu.VMEM((CHUNK, d), jnp.float32),
            pltpu.VMEM_SHARED((nacc, d), jnp.float32),
            pltpu.SemaphoreType.DMA,
        ],
        compiler_params=pltpu.CompilerParams(needs_layout_passes=False),
    )
    def agg_kernel(src_hbm, dst_hbm, g_hbm, out_hbm, src_v, dst_v, rows_v,
                   acc_sh, sem):
        c = lax.axis_index("c")
        s = lax.axis_index("s")
        wid = s * NC + c
        zeros16 = jnp.zeros((LANES,), jnp.float32)

        def zero_rows(i, _):
            for j in range(d // LANES):
                rows_v[i, pl.ds(j * LANES, LANES)] = zeros16
            return 0

        lax.fori_loop(0, CHUNK, zero_rows, 0)
        base = s * rows_per_tile

        def zero_acc(k, _):
            pltpu.sync_copy(rows_v, acc_sh.at[pl.ds(base + k * CHUNK, CHUNK)])
            return 0

        lax.fori_loop(0, rows_per_tile // CHUNK, zero_acc, 0)
        rem = rows_per_tile % CHUNK
        if rem:
            pltpu.sync_copy(
                rows_v.at[pl.ds(0, rem)],
                acc_sh.at[pl.ds(base + (rows_per_tile // CHUNK) * CHUNK, rem)],
            )
        plsc.subcore_barrier()

        pltpu.sync_copy(src_hbm.at[wid], src_v)
        pltpu.sync_copy(dst_hbm.at[wid], dst_v)

        def chunk_body(j, _):
            pltpu.async_copy(g_hbm.at[src_v.at[j]], rows_v, sem).wait()
            pltpu.sync_copy(rows_v, acc_sh.at[dst_v.at[j]], add=True)
            return 0

        lax.fori_loop(0, n_chunks, chunk_body, 0)
        plsc.subcore_barrier()
        pltpu.sync_copy(acc_sh.at[pl.ds(base, rows_per_tile)],
                        out_hbm.at[c, pl.ds(base, rows_per_tile)])

    return agg_kernel


def _g_body(x_ref, w_ref, degp_ref, g_ref):
    deg = jnp.sum(degp_ref[...], axis=1, keepdims=True) + 1.0
    dinv = lax.rsqrt(deg)
    h = jnp.dot(x_ref[...], w_ref[...], preferred_element_type=jnp.float32)
    g_ref[...] = h * dinv


def _out_body(acc_ref, g_ref, degp_ref, b_ref, o_ref):
    deg = jnp.sum(degp_ref[...], axis=1, keepdims=True) + 1.0
    dinv = lax.rsqrt(deg)
    a = acc_ref[0] + acc_ref[1]
    o_ref[...] = dinv * (a + g_ref[...]) + b_ref[...]


def kernel(x, edge_index, W, b):
    n, d_in = x.shape
    d_out = W.shape[1]
    e = edge_index.shape[1]

    src = edge_index[0].astype(jnp.int32)
    dst = edge_index[1].astype(jnp.int32)

    npad = ((n + LANES - 1) // LANES) * LANES
    rpt = (((n + 1 + NS - 1) // NS + 7) // 8) * 8
    nacc = rpt * NS
    e_t = e // NW
    ept = ((e + NW * CHUNK - 1) // (NW * CHUNK)) * CHUNK
    n_chunks = ept // CHUNK
    n_pad_edges = ept * NW - e

    srcp = jnp.concatenate(
        [src, jnp.zeros((n_pad_edges,), jnp.int32)]).reshape(NW, n_chunks, CHUNK)
    dstp = jnp.concatenate(
        [dst, jnp.full((n_pad_edges,), n, jnp.int32)]).reshape(NW, n_chunks, CHUNK)

    degp = _make_deg_kernel(npad, e_t)(dst)
    degp_t = jnp.transpose(degp)[:n]

    bn = 2000
    grid = (n // bn,)
    g = pl.pallas_call(
        _g_body,
        grid=grid,
        in_specs=[
            pl.BlockSpec((bn, d_in), lambda i: (i, 0)),
            pl.BlockSpec((d_in, d_out), lambda i: (0, 0)),
            pl.BlockSpec((bn, NW), lambda i: (i, 0)),
        ],
        out_specs=pl.BlockSpec((bn, d_out), lambda i: (i, 0)),
        out_shape=jax.ShapeDtypeStruct((n, d_out), jnp.float32),
    )(x, W, degp_t)

    acc = _make_agg_kernel(nacc, d_out, n_chunks)(srcp, dstp, g)

    out = pl.pallas_call(
        _out_body,
        grid=grid,
        in_specs=[
            pl.BlockSpec((NC, bn, d_out), lambda i: (0, i, 0)),
            pl.BlockSpec((bn, d_out), lambda i: (i, 0)),
            pl.BlockSpec((bn, NW), lambda i: (i, 0)),
            pl.BlockSpec((1, d_out), lambda i: (0, 0)),
        ],
        out_specs=pl.BlockSpec((bn, d_out), lambda i: (i, 0)),
        out_shape=jax.ShapeDtypeStruct((n, d_out), jnp.float32),
    )(acc, g, degp_t, b.reshape(1, d_out))
    return out
